# baseline (speedup 1.0000x reference)
import functools

import jax
import jax.numpy as jnp
from jax import lax
from jax.experimental import pallas as pl
from jax.experimental.pallas import tpu as pltpu
from jax.experimental.pallas import tpu_sc as plsc

_LN2 = 0.6931471805599453
_C = (3.3869653018518764e-08, 0.9999942724811797, -0.4998385618342408,
      0.33154861652015205, -0.23982616050152822, 0.16582275268624033,
      -0.09325203898173497, 0.03484971247629039, -0.006151470961286831)

_NW = 32
_LANES = 16


def _rgather(vec, idx):
    return lax.gather(
        vec, idx[:, None],
        dimension_numbers=lax.GatherDimensionNumbers(
            offset_dims=(), collapsed_slice_dims=(0,), start_index_map=(0,)),
        slice_sizes=(1,),
        mode=lax.GatherScatterMode.PROMISE_IN_BOUNDS)


def _log_1_4(x):
    big = x >= 2.0
    m = jnp.where(big, x * 0.5, x)
    e = jnp.where(big, _LN2, 0.0)
    t = m - 1.0
    p = jnp.full((_LANES,), _C[8], jnp.float32)
    for c in (_C[7], _C[6], _C[5], _C[4], _C[3], _C[2], _C[1], _C[0]):
        p = p * t + c
    return e + p


def _make_sc(B, L, rpc):
    rows = B * L
    rps = rows // _NW
    nchunk = rps // rpc
    mesh = plsc.VectorSubcoreMesh(core_axis_name="c", subcore_axis_name="s")

    @functools.partial(
        pl.kernel, mesh=mesh,
        out_type=jax.ShapeDtypeStruct((rows,), jnp.float32),
        scratch_types=[
            pltpu.VMEM((rpc * L,), jnp.float32),
            pltpu.VMEM((rpc * L,), jnp.float32),
            pltpu.VMEM((rpc * L // 2,), jnp.int32),
            pltpu.VMEM((L,), jnp.float32),
            pltpu.VMEM((rps,), jnp.float32),
        ],
    )
    def sc(sb_hbm, se_hbm, pr_hbm, prd_hbm, out_hbm, sbv, sev, prv, wv, rs):
        wid = lax.axis_index("s") * 2 + lax.axis_index("c")
        row0 = wid * rps
        b = row0 // L
        pltpu.sync_copy(prd_hbm.at[pl.ds(pl.multiple_of(b * L, L), L)], wv)
        iota = lax.iota(jnp.int32, _LANES)
        half = iota >> 1
        shamt = (iota & 1) * 16

        def chunk_body(c, carry):
            base = pl.multiple_of(row0 * L + c * (rpc * L), rpc * L)
            half_base = pl.multiple_of((row0 * L) // 2 + c * (rpc * L // 2),
                                       rpc * L // 2)
            pltpu.sync_copy(sb_hbm.at[pl.ds(base, rpc * L)], sbv)
            pltpu.sync_copy(se_hbm.at[pl.ds(base, rpc * L)], sev)
            pltpu.sync_copy(pr_hbm.at[pl.ds(half_base, rpc * L // 2)], prv)

            def row_body(r, rsvec):
                def jp_body(jp, acc):
                    w16 = prv[pl.ds(r * (L // 2) + jp * _LANES, _LANES)]
                    for a in (0, 1):
                        j0 = jp * 2 * _LANES + a * _LANES
                        off = r * L + j0
                        sbx = sbv[pl.ds(off, _LANES)]
                        sex = sev[pl.ds(off, _LANES)]
                        w = _rgather(w16, a * 8 + half)
                        gb = ((w >> shamt) & 1).astype(jnp.float32)
                        ge = ((w >> (shamt + 8)) & 1).astype(jnp.float32)
                        wj = wv[pl.ds(j0, _LANES)]
                        u = jnp.exp(-jnp.abs(sbx))
                        v = jnp.exp(-jnp.abs(sex))
                        splog = _log_1_4((1.0 + u) * (1.0 + v))
                        nl = (jnp.maximum(sbx, 0.0) + jnp.maximum(sex, 0.0)
                              + splog - gb * sbx - ge * sex)
                        acc = acc + wj * nl
                    return acc

                acc = lax.fori_loop(0, L // (2 * _LANES), jp_body,
                                    jnp.zeros((_LANES,), jnp.float32))
                for sh in (8, 4, 2, 1):
                    acc = acc + _rgather(acc, iota ^ sh)
                return jnp.where(iota == r, acc, rsvec)

            rsvec = lax.fori_loop(0, rpc, row_body,
                                  jnp.zeros((_LANES,), jnp.float32))
            rs[pl.ds(c * rpc, _LANES)] = rsvec
            return carry

        lax.fori_loop(0, nchunk, chunk_body, 0)
        pltpu.sync_copy(rs, out_hbm.at[pl.ds(pl.multiple_of(row0, rps), rps)])

    return sc


def kernel(s_arg_begin, s_arg_end, props, prd_mask, arg_begin_mask, arg_end_mask):
    B, L, _ = s_arg_begin.shape
    n = B * L * L
    sb_flat = s_arg_begin.reshape(n)
    se_flat = s_arg_end.reshape(n)
    pr32 = jax.lax.bitcast_convert_type(
        props.view(jnp.uint8).reshape(n // 2, 4), jnp.int32)
    prd_f = prd_mask.astype(jnp.float32)
    row_sums = _make_sc(B, L, rpc=16)(sb_flat, se_flat, pr32, prd_f.reshape(-1))
    denom = jnp.sum(jnp.sum(prd_f, axis=1) ** 2)
    return jnp.dot(row_sums, prd_f.reshape(-1)) / denom


# speedup vs baseline: 5.6794x; 5.6794x over previous
import functools

import jax
import jax.numpy as jnp
from jax import lax
from jax.experimental import pallas as pl
from jax.experimental.pallas import tpu as pltpu
from jax.experimental.pallas import tpu_sc as plsc

_LN2 = 0.6931471805599453
_C = (3.3869653018518764e-08, 0.9999942724811797, -0.4998385618342408,
      0.33154861652015205, -0.23982616050152822, 0.16582275268624033,
      -0.09325203898173497, 0.03484971247629039, -0.006151470961286831)

_NW = 32
_LANES = 16


def _rgather(vec, idx):
    return lax.gather(
        vec, idx[:, None],
        dimension_numbers=lax.GatherDimensionNumbers(
            offset_dims=(), collapsed_slice_dims=(0,), start_index_map=(0,)),
        slice_sizes=(1,),
        mode=lax.GatherScatterMode.PROMISE_IN_BOUNDS)


def _log_1_4(x):
    big = x >= 2.0
    m = jnp.where(big, x * 0.5, x)
    e = jnp.where(big, _LN2, 0.0)
    t = m - 1.0
    p = jnp.full((_LANES,), _C[8], jnp.float32)
    for c in (_C[7], _C[6], _C[5], _C[4], _C[3], _C[2], _C[1], _C[0]):
        p = p * t + c
    return e + p


def _make_sc(B, L, rpc):
    rows = B * L
    rps = rows // _NW
    nchunk = rps // rpc
    sub_per_b = L // rps
    mesh = plsc.VectorSubcoreMesh(core_axis_name="c", subcore_axis_name="s")

    @functools.partial(
        pl.kernel, mesh=mesh,
        out_type=jax.ShapeDtypeStruct((rows,), jnp.float32),
        scratch_types=[
            pltpu.VMEM((rpc, L), jnp.float32),
            pltpu.VMEM((rpc, L), jnp.float32),
            pltpu.VMEM((rpc, L // 2), jnp.int32),
            pltpu.VMEM((L,), jnp.float32),
            pltpu.VMEM((rps,), jnp.float32),
        ],
    )
    def sc(sb_hbm, se_hbm, pr_hbm, prd_hbm, out_hbm, sbv, sev, prv, wv, rs):
        wid = lax.axis_index("s") * 2 + lax.axis_index("c")
        row0 = wid * rps
        b = row0 // L
        i_base = (wid % sub_per_b) * rps
        pltpu.sync_copy(prd_hbm.at[b], wv)
        iota = lax.iota(jnp.int32, _LANES)
        half = iota >> 1
        shamt = (iota & 1) * 16

        def chunk_body(c, rsvec):
            i0 = pl.multiple_of(i_base + c * rpc, rpc)
            pltpu.sync_copy(sb_hbm.at[b, pl.ds(i0, rpc), :], sbv)
            pltpu.sync_copy(se_hbm.at[b, pl.ds(i0, rpc), :], sev)
            pltpu.sync_copy(pr_hbm.at[b, pl.ds(i0, rpc), :], prv)

            def jp_body(jp, accs):
                out = []
                for r in range(rpc):
                    w16 = prv[r, pl.ds(jp * _LANES, _LANES)]
                    acc = accs[r]
                    for a in (0, 1):
                        j0 = jp * 2 * _LANES + a * _LANES
                        sbx = sbv[r, pl.ds(j0, _LANES)]
                        sex = sev[r, pl.ds(j0, _LANES)]
                        w = _rgather(w16, a * 8 + half)
                        gb = ((w >> shamt) & 1).astype(jnp.float32)
                        ge = ((w >> (shamt + 8)) & 1).astype(jnp.float32)
                        wj = wv[pl.ds(j0, _LANES)]
                        u = jnp.exp(-jnp.abs(sbx))
                        v = jnp.exp(-jnp.abs(sex))
                        splog = _log_1_4((1.0 + u) * (1.0 + v))
                        nl = (jnp.maximum(sbx, 0.0) + jnp.maximum(sex, 0.0)
                              + splog - gb * sbx - ge * sex)
                        acc = acc + wj * nl
                    out.append(acc)
                return tuple(out)

            zero = jnp.zeros((_LANES,), jnp.float32)
            accs = lax.fori_loop(0, L // (2 * _LANES), jp_body, (zero,) * rpc)
            lane_off = (c % 2) * rpc
            for r in range(rpc):
                acc = accs[r]
                for sh in (8, 4, 2, 1):
                    acc = acc + _rgather(acc, iota ^ sh)
                rsvec = jnp.where(iota == lane_off + r, acc, rsvec)
            store_vec = rsvec

            @pl.when(c % 2 == 1)
            def _store():
                rs[pl.ds((c // 2) * _LANES, _LANES)] = store_vec

            return rsvec

        lax.fori_loop(0, nchunk, chunk_body,
                      jnp.zeros((_LANES,), jnp.float32))
        pltpu.sync_copy(rs, out_hbm.at[pl.ds(pl.multiple_of(row0, rps), rps)])

    return sc


def kernel(s_arg_begin, s_arg_end, props, prd_mask, arg_begin_mask, arg_end_mask):
    B, L, _ = s_arg_begin.shape
    pr32 = jax.lax.bitcast_convert_type(
        props.view(jnp.uint8).reshape(B, L, L // 2, 4), jnp.int32)
    prd_f = prd_mask.astype(jnp.float32)
    row_sums = _make_sc(B, L, rpc=8)(s_arg_begin, s_arg_end, pr32, prd_f)
    denom = jnp.sum(jnp.sum(prd_f, axis=1) ** 2)
    return jnp.dot(row_sums, prd_f.reshape(-1)) / denom


# speedup vs baseline: 11.6254x; 2.0470x over previous
import functools

import jax
import jax.numpy as jnp
from jax import lax
from jax.experimental import pallas as pl
from jax.experimental.pallas import tpu as pltpu
from jax.experimental.pallas import tpu_sc as plsc

_LN2 = 0.6931471805599453
_LOG2E = 1.4426950408889634
_C = (3.3869653018518764e-08, 0.9999942724811797, -0.4998385618342408,
      0.33154861652015205, -0.23982616050152822, 0.16582275268624033,
      -0.09325203898173497, 0.03484971247629039, -0.006151470961286831)

_NW = 32
_LANES = 16
_B_SC = 4


def _tc_body(sb_ref, se_ref, pr_ref, rows_ref, cols_ref, out_ref):
    @pl.when((pl.program_id(0) == 0) & (pl.program_id(1) == 0))
    def _init():
        out_ref[...] = jnp.zeros_like(out_ref)

    sb = sb_ref[0]
    se = se_ref[0]
    f = pr_ref[0].astype(jnp.float32)
    ge = jnp.floor(f * (1.0 / 256.0))
    gb = f - 256.0 * ge
    pad = rows_ref[0] * cols_ref[0]
    u = lax.exp2(-jnp.abs(sb) * _LOG2E)
    v = lax.exp2(-jnp.abs(se) * _LOG2E)
    splog = jnp.log2((1.0 + u) * (1.0 + v)) * _LN2
    sp_sum = jnp.maximum(sb, 0.0) + jnp.maximum(se, 0.0) + splog
    neg_logp = sp_sum - gb * sb - ge * se
    out_ref[...] += jnp.sum(neg_logp * pad, axis=0, keepdims=True)


def _tc_partial(sb, se, pr16, rows, cols, b_tc, Lb=256):
    B, L, _ = sb.shape
    grid = (b_tc, L // Lb)
    return pl.pallas_call(
        _tc_body,
        grid=grid,
        in_specs=[
            pl.BlockSpec((1, Lb, L), lambda b, i: (b, i, 0)),
            pl.BlockSpec((1, Lb, L), lambda b, i: (b, i, 0)),
            pl.BlockSpec((1, Lb, L), lambda b, i: (b, i, 0)),
            pl.BlockSpec((1, Lb, 1), lambda b, i: (b, i, 0)),
            pl.BlockSpec((1, 1, L), lambda b, i: (b, 0, 0)),
        ],
        out_specs=pl.BlockSpec((1, L), lambda b, i: (0, 0)),
        out_shape=jax.ShapeDtypeStruct((1, L), jnp.float32),
    )(sb, se, pr16, rows, cols)


def _rgather(vec, idx):
    return lax.gather(
        vec, idx[:, None],
        dimension_numbers=lax.GatherDimensionNumbers(
            offset_dims=(), collapsed_slice_dims=(0,), start_index_map=(0,)),
        slice_sizes=(1,),
        mode=lax.GatherScatterMode.PROMISE_IN_BOUNDS)


def _log_1_4(x):
    big = x >= 2.0
    m = jnp.where(big, x * 0.5, x)
    e = jnp.where(big, _LN2, 0.0)
    t = m - 1.0
    p = jnp.full((_LANES,), _C[8], jnp.float32)
    for c in (_C[7], _C[6], _C[5], _C[4], _C[3], _C[2], _C[1], _C[0]):
        p = p * t + c
    return e + p


def _make_sc(L, b0, nb, rpc):
    rows = nb * L
    rps = rows // _NW
    nchunk = rps // rpc
    sub_per_b = L // rps
    mesh = plsc.VectorSubcoreMesh(core_axis_name="c", subcore_axis_name="s")

    @functools.partial(
        pl.kernel, mesh=mesh,
        out_type=jax.ShapeDtypeStruct((rows,), jnp.float32),
        scratch_types=[
            pltpu.VMEM((rpc, L), jnp.float32),
            pltpu.VMEM((rpc, L), jnp.float32),
            pltpu.VMEM((rpc, L // 2), jnp.int32),
            pltpu.VMEM((L,), jnp.float32),
            pltpu.VMEM((rps,), jnp.float32),
        ],
    )
    def sc(sb_hbm, se_hbm, pr_hbm, prd_hbm, out_hbm, sbv, sev, prv, wv, rs):
        wid = lax.axis_index("s") * 2 + lax.axis_index("c")
        row0 = wid * rps
        b = b0 + row0 // L
        i_base = (wid % sub_per_b) * rps
        pltpu.sync_copy(prd_hbm.at[b], wv)
        iota = lax.iota(jnp.int32, _LANES)
        half = iota >> 1
        shamt = (iota & 1) * 16

        def chunk_body(c, rsvec):
            i0 = pl.multiple_of(i_base + c * rpc, rpc)
            pltpu.sync_copy(sb_hbm.at[b, pl.ds(i0, rpc), :], sbv)
            pltpu.sync_copy(se_hbm.at[b, pl.ds(i0, rpc), :], sev)
            pltpu.sync_copy(pr_hbm.at[b, pl.ds(i0, rpc), :], prv)

            def jp_body(jp, accs):
                out = []
                for r in range(rpc):
                    w16 = prv[r, pl.ds(jp * _LANES, _LANES)]
                    acc = accs[r]
                    for a in (0, 1):
                        j0 = jp * 2 * _LANES + a * _LANES
                        sbx = sbv[r, pl.ds(j0, _LANES)]
                        sex = sev[r, pl.ds(j0, _LANES)]
                        w = _rgather(w16, a * 8 + half)
                        gb = ((w >> shamt) & 1).astype(jnp.float32)
                        ge = ((w >> (shamt + 8)) & 1).astype(jnp.float32)
                        wj = wv[pl.ds(j0, _LANES)]
                        u = jnp.exp(-jnp.abs(sbx))
                        v = jnp.exp(-jnp.abs(sex))
                        splog = _log_1_4((1.0 + u) * (1.0 + v))
                        nl = (jnp.maximum(sbx, 0.0) + jnp.maximum(sex, 0.0)
                              + splog - gb * sbx - ge * sex)
                        acc = acc + wj * nl
                    out.append(acc)
                return tuple(out)

            zero = jnp.zeros((_LANES,), jnp.float32)
            accs = lax.fori_loop(0, L // (2 * _LANES), jp_body, (zero,) * rpc)
            lane_off = (c % 2) * rpc
            for r in range(rpc):
                acc = accs[r]
                for sh in (8, 4, 2, 1):
                    acc = acc + _rgather(acc, iota ^ sh)
                rsvec = jnp.where(iota == lane_off + r, acc, rsvec)
            store_vec = rsvec

            @pl.when(c % 2 == 1)
            def _store():
                rs[pl.ds((c // 2) * _LANES, _LANES)] = store_vec

            return rsvec

        lax.fori_loop(0, nchunk, chunk_body,
                      jnp.zeros((_LANES,), jnp.float32))
        pltpu.sync_copy(rs, out_hbm.at[pl.ds(pl.multiple_of(row0, rps), rps)])

    return sc


def kernel(s_arg_begin, s_arg_end, props, prd_mask, arg_begin_mask, arg_end_mask):
    B, L, _ = s_arg_begin.shape
    b_tc = B - _B_SC
    pr16 = jax.lax.bitcast_convert_type(props.view(jnp.uint8), jnp.uint16)
    pr32 = jax.lax.bitcast_convert_type(
        props.view(jnp.uint8).reshape(B, L, L // 2, 4), jnp.int32)
    prd_f = prd_mask.astype(jnp.float32)
    rows = prd_f[:, :, None]
    cols = prd_f[:, None, :]

    sc_rows = _make_sc(L, b_tc, _B_SC, rpc=8)(
        s_arg_begin, s_arg_end, pr32, prd_f)
    num_tc = _tc_partial(s_arg_begin, s_arg_end, pr16, rows, cols, b_tc)

    num = jnp.sum(num_tc) + jnp.dot(sc_rows, prd_f[b_tc:].reshape(-1))
    denom = jnp.sum(jnp.sum(prd_f, axis=1) ** 2)
    return num / denom


# speedup vs baseline: 21.7311x; 1.8693x over previous
import jax
import jax.numpy as jnp
from jax.experimental import pallas as pl


def _body(sb_ref, se_ref, pr_ref, rows_ref, cols_ref, out_ref):
    @pl.when((pl.program_id(0) == 0) & (pl.program_id(1) == 0))
    def _init():
        out_ref[...] = jnp.zeros_like(out_ref)

    sb = sb_ref[0]
    se = se_ref[0]
    f = pr_ref[0].astype(jnp.float32)
    ge = jnp.floor(f * (1.0 / 256.0))
    gb = f - 256.0 * ge
    pad = rows_ref[0] * cols_ref[0]
    log2e = 1.4426950408889634
    ln2 = 0.6931471805599453
    u = jax.lax.exp2(-jnp.abs(sb) * log2e)
    v = jax.lax.exp2(-jnp.abs(se) * log2e)
    splog = jnp.log2((1.0 + u) * (1.0 + v)) * ln2
    sp_sum = jnp.maximum(sb, 0.0) + jnp.maximum(se, 0.0) + splog
    neg_logp = sp_sum - gb * sb - ge * se
    out_ref[...] += jnp.sum(neg_logp * pad, axis=0, keepdims=True)


def kernel(s_arg_begin, s_arg_end, props, prd_mask, arg_begin_mask, arg_end_mask):
    B, L, _ = s_arg_begin.shape
    pr16 = jax.lax.bitcast_convert_type(props.view(jnp.uint8), jnp.uint16)
    prd_f = prd_mask.astype(jnp.float32)
    rows = prd_f[:, :, None]
    cols = prd_f[:, None, :]
    Lb = 128
    grid = (B, L // Lb)
    num = pl.pallas_call(
        _body,
        grid=grid,
        in_specs=[
            pl.BlockSpec((1, Lb, L), lambda b, i: (b, i, 0)),
            pl.BlockSpec((1, Lb, L), lambda b, i: (b, i, 0)),
            pl.BlockSpec((1, Lb, L), lambda b, i: (b, i, 0)),
            pl.BlockSpec((1, Lb, 1), lambda b, i: (b, i, 0)),
            pl.BlockSpec((1, 1, L), lambda b, i: (b, 0, 0)),
        ],
        out_specs=pl.BlockSpec((1, L), lambda b, i: (0, 0)),
        out_shape=jax.ShapeDtypeStruct((1, L), jnp.float32),
    )(s_arg_begin, s_arg_end, pr16, rows, cols)
    denom = jnp.sum(jnp.sum(prd_f, axis=1) ** 2)
    return jnp.sum(num) / denom


# speedup vs baseline: 25.9041x; 1.1920x over previous
import jax
import jax.numpy as jnp
from jax.experimental import pallas as pl


def _body(sb_ref, se_ref, pr_ref, rows_ref, cols_ref, out_ref):
    @pl.when((pl.program_id(0) == 0) & (pl.program_id(1) == 0))
    def _init():
        out_ref[...] = jnp.zeros_like(out_ref)

    sb = sb_ref[0]
    se = se_ref[0]
    f = pr_ref[0].astype(jnp.float32)
    ge = jnp.floor(f * (1.0 / 256.0))
    gb = f - 256.0 * ge
    pad = rows_ref[0] * cols_ref[0]
    log2e = 1.4426950408889634
    ln2 = 0.6931471805599453
    u = jax.lax.exp2(-jnp.abs(sb) * log2e)
    v = jax.lax.exp2(-jnp.abs(se) * log2e)
    splog = jnp.log2((1.0 + u) * (1.0 + v)) * ln2
    sp_sum = jnp.maximum(sb, 0.0) + jnp.maximum(se, 0.0) + splog
    neg_logp = sp_sum - gb * sb - ge * se
    out_ref[...] += jnp.sum(neg_logp * pad, axis=0, keepdims=True)


def kernel(s_arg_begin, s_arg_end, props, prd_mask, arg_begin_mask, arg_end_mask):
    B, L, _ = s_arg_begin.shape
    pr16 = jax.lax.bitcast_convert_type(props.view(jnp.uint8), jnp.uint16)
    prd_f = prd_mask.astype(jnp.float32)
    rows = prd_f[:, :, None]
    cols = prd_f[:, None, :]
    Lb = 512
    grid = (B, L // Lb)
    num = pl.pallas_call(
        _body,
        grid=grid,
        in_specs=[
            pl.BlockSpec((1, Lb, L), lambda b, i: (b, i, 0)),
            pl.BlockSpec((1, Lb, L), lambda b, i: (b, i, 0)),
            pl.BlockSpec((1, Lb, L), lambda b, i: (b, i, 0)),
            pl.BlockSpec((1, Lb, 1), lambda b, i: (b, i, 0)),
            pl.BlockSpec((1, 1, L), lambda b, i: (b, 0, 0)),
        ],
        out_specs=pl.BlockSpec((1, L), lambda b, i: (0, 0)),
        out_shape=jax.ShapeDtypeStruct((1, L), jnp.float32),
    )(s_arg_begin, s_arg_end, pr16, rows, cols)
    denom = jnp.sum(jnp.sum(prd_f, axis=1) ** 2)
    return jnp.sum(num) / denom
